# baseline (speedup 1.0000x reference)
import jax
import jax.numpy as jnp
from jax import lax
from jax.experimental import pallas as pl
from jax.experimental.pallas import tpu as pltpu
from jax.experimental.pallas import tpu_sc as plsc

NC = 2
NS = 16
NW = NC * NS
BATCH = 128
CH_B = 4
DEG_L = 16
ZCH = 32
NRING = 4
SC0_SHARE = 0.65


def _nbw0(per_pair):
    return int(round(per_pair * SC0_SHARE / (2 * CH_B))) * 2 * CH_B


def _sc_agg_body(x_hbm, send_hbm, recv_hbm, zrow_hbm, zdeg_hbm, ones_hbm,
                 sum_out, deg_out,
                 send_v0, send_v1, recv_v0, recv_v1,
                 rows0, rows1, rows2, rows3, ones_v, zrow_v, zdeg_v,
                 sum_sh, deg_sh,
                 gsem0, gsem1, gsem2, gsem3,
                 ssem0, ssem1, ssem2, ssem3, dsem, isem):
    per_pair = send_hbm.shape[0] // NS
    nbw0 = _nbw0(per_pair)
    nbw1 = per_pair - nbw0
    rt = sum_sh.shape[0] // NS
    c = lax.axis_index("c")
    s = lax.axis_index("s")
    w0 = s * nbw0 + c * (NS * nbw0 + s * (nbw1 - nbw0))
    nch = (nbw0 + c * (nbw1 - nbw0)) // CH_B

    pltpu.sync_copy(zrow_hbm, zrow_v)
    pltpu.sync_copy(zdeg_hbm, zdeg_v)
    pltpu.sync_copy(ones_hbm, ones_v)

    f0 = pltpu.async_copy(send_hbm.at[pl.ds(w0, CH_B), :], send_v0, isem)
    f1 = pltpu.async_copy(recv_hbm.at[pl.ds(w0, CH_B), :], recv_v0, isem)
    zd = []
    for k in range(rt // ZCH):
        zd.append(pltpu.async_copy(
            zrow_v, sum_sh.at[pl.ds(s * rt + k * ZCH, ZCH), :], ssem0))
        zd.append(pltpu.async_copy(
            zdeg_v, deg_sh.at[pl.ds(s * rt + k * ZCH, ZCH), :], ssem1))

    rows = (rows0, rows1, rows2, rows3)
    gsem = (gsem0, gsem1, gsem2, gsem3)
    ssem = (ssem0, ssem1, ssem2, ssem3)

    f0.wait()
    f1.wait()
    gd_pre = []
    for j in range(NRING - 1):
        gd_pre.append(
            pltpu.async_copy(x_hbm.at[send_v0.at[j]], rows[j], gsem[j]))
    for zz in zd:
        zz.wait()
    plsc.subcore_barrier()

    def process_chunk(sv, rv, pre=None):
        gd = [None] * NRING
        sd = [None] * NRING
        dd = []
        for j in range(NRING - 1):
            gd[j] = pre[j] if pre is not None else pltpu.async_copy(
                x_hbm.at[sv.at[j]], rows[j], gsem[j])
        for j in range(CH_B):
            p = j % NRING
            if j + NRING - 1 < CH_B:
                q = (j + NRING - 1) % NRING
                if sd[q] is not None:
                    sd[q].wait()
                gd[q] = pltpu.async_copy(
                    x_hbm.at[sv.at[j + NRING - 1]], rows[q], gsem[q])
            gd[p].wait()
            sd[p] = pltpu.async_copy(
                rows[p], sum_sh.at[rv.at[j]], ssem[p], add=True)
            dd.append(pltpu.async_copy(
                ones_v, deg_sh.at[rv.at[j]], dsem, add=True))
        for p in range(NRING):
            if sd[p] is not None:
                sd[p].wait()
        for d_ in dd:
            d_.wait()

    last = w0 + (nch - 1) * CH_B

    fb0 = pltpu.async_copy(
        send_hbm.at[pl.ds(w0 + CH_B, CH_B), :], send_v1, isem)
    fb1 = pltpu.async_copy(
        recv_hbm.at[pl.ds(w0 + CH_B, CH_B), :], recv_v1, isem)
    process_chunk(send_v0, recv_v0, pre=gd_pre)
    fb0.wait()
    fb1.wait()
    nbase = jnp.minimum(w0 + 2 * CH_B, last)
    fa0 = pltpu.async_copy(send_hbm.at[pl.ds(nbase, CH_B), :], send_v0, isem)
    fa1 = pltpu.async_copy(recv_hbm.at[pl.ds(nbase, CH_B), :], recv_v0, isem)
    process_chunk(send_v1, recv_v1)
    fa0.wait()
    fa1.wait()

    def iter2(i, carry):
        base = w0 + (2 + 2 * i) * CH_B
        fb0 = pltpu.async_copy(
            send_hbm.at[pl.ds(base + CH_B, CH_B), :], send_v1, isem)
        fb1 = pltpu.async_copy(
            recv_hbm.at[pl.ds(base + CH_B, CH_B), :], recv_v1, isem)
        process_chunk(send_v0, recv_v0)
        fb0.wait()
        fb1.wait()
        nbase = jnp.minimum(base + 2 * CH_B, last)
        fa0 = pltpu.async_copy(
            send_hbm.at[pl.ds(nbase, CH_B), :], send_v0, isem)
        fa1 = pltpu.async_copy(
            recv_hbm.at[pl.ds(nbase, CH_B), :], recv_v0, isem)
        process_chunk(send_v1, recv_v1)
        fa0.wait()
        fa1.wait()
        return carry

    lax.fori_loop(0, (nch - 2) // 2, iter2, 0)
    plsc.subcore_barrier()

    pltpu.sync_copy(sum_sh.at[pl.ds(s * rt, rt), :],
                    sum_out.at[c, pl.ds(s * rt, rt), :])
    pltpu.sync_copy(deg_sh.at[pl.ds(s * rt, rt), :],
                    deg_out.at[c, pl.ds(s * rt, rt), :])


def _sc_aggregate(xb, send2d, recv2d, r_rows):
    d = xb.shape[1]
    zrow = jnp.zeros((ZCH, d), jnp.bfloat16)
    zdeg = jnp.zeros((ZCH, DEG_L), jnp.float32)
    ones = jnp.ones((BATCH, DEG_L), jnp.float32)

    mesh = plsc.VectorSubcoreMesh(core_axis_name="c", subcore_axis_name="s")
    f = pl.kernel(
        _sc_agg_body,
        out_type=(
            jax.ShapeDtypeStruct((NC, r_rows, d), jnp.bfloat16),
            jax.ShapeDtypeStruct((NC, r_rows, DEG_L), jnp.float32),
        ),
        mesh=mesh,
        scratch_types=(
            pltpu.VMEM((CH_B, BATCH), jnp.int32),
            pltpu.VMEM((CH_B, BATCH), jnp.int32),
            pltpu.VMEM((CH_B, BATCH), jnp.int32),
            pltpu.VMEM((CH_B, BATCH), jnp.int32),
            pltpu.VMEM((BATCH, d), jnp.bfloat16),
            pltpu.VMEM((BATCH, d), jnp.bfloat16),
            pltpu.VMEM((BATCH, d), jnp.bfloat16),
            pltpu.VMEM((BATCH, d), jnp.bfloat16),
            pltpu.VMEM((BATCH, DEG_L), jnp.float32),
            pltpu.VMEM((ZCH, d), jnp.bfloat16),
            pltpu.VMEM((ZCH, DEG_L), jnp.float32),
            pltpu.VMEM_SHARED((r_rows, d), jnp.bfloat16),
            pltpu.VMEM_SHARED((r_rows, DEG_L), jnp.float32),
            pltpu.SemaphoreType.DMA,
            pltpu.SemaphoreType.DMA,
            pltpu.SemaphoreType.DMA,
            pltpu.SemaphoreType.DMA,
            pltpu.SemaphoreType.DMA,
            pltpu.SemaphoreType.DMA,
            pltpu.SemaphoreType.DMA,
            pltpu.SemaphoreType.DMA,
            pltpu.SemaphoreType.DMA,
            pltpu.SemaphoreType.DMA,
        ),
        compiler_params=pltpu.CompilerParams(use_tc_tiling_on_sc=False),
    )
    return f(xb, send2d, recv2d, zrow, zdeg, ones)


def _finalize_body(x_ref, sum_ref, deg_ref, wsT_ref, wnT_ref, b_ref,
                   scale_ref, out_ref):
    xb = x_ref[...]
    sb = sum_ref[0].astype(jnp.float32) + sum_ref[1].astype(jnp.float32)
    db = deg_ref[0, :, 0:1] + deg_ref[1, :, 0:1]
    recip = jnp.where(db > 0, scale_ref[0] / db, 0.0)
    agg = sb * recip
    acc = lax.dot_general(xb, wsT_ref[...], (((1,), (0,)), ((), ())),
                          precision=lax.Precision.HIGHEST,
                          preferred_element_type=jnp.float32)
    acc = acc + lax.dot_general(agg, wnT_ref[...], (((1,), (0,)), ((), ())),
                                precision=lax.Precision.HIGHEST,
                                preferred_element_type=jnp.float32)
    out_ref[...] = acc + b_ref[...]


def _finalize(x, sum_p, deg_p, w_self, b_self, w_neigh, scale):
    n, d = x.shape
    bn = 1000
    grid = (n // bn,)
    return pl.pallas_call(
        _finalize_body,
        grid=grid,
        in_specs=[
            pl.BlockSpec((bn, d), lambda i: (i, 0)),
            pl.BlockSpec((NC, bn, d), lambda i: (0, i, 0)),
            pl.BlockSpec((NC, bn, DEG_L), lambda i: (0, i, 0)),
            pl.BlockSpec((d, d), lambda i: (0, 0)),
            pl.BlockSpec((d, d), lambda i: (0, 0)),
            pl.BlockSpec((1, d), lambda i: (0, 0)),
            pl.BlockSpec(memory_space=pltpu.SMEM),
        ],
        out_specs=pl.BlockSpec((bn, d), lambda i: (i, 0)),
        out_shape=jax.ShapeDtypeStruct((n, d), jnp.float32),
    )(x, sum_p, deg_p, w_self.T, w_neigh.T, b_self.reshape(1, d), scale)


def kernel(x, senders, receivers, w_self, b_self, w_neigh, zero_edges):
    n, d = x.shape
    e = senders.shape[0]
    senders = senders.astype(jnp.int32)
    receivers = receivers.astype(jnp.int32)

    per_pair = -(-(-(-e // BATCH)) // (NS * 2 * CH_B)) * 2 * CH_B
    nb_total = NS * per_pair
    pad_e = nb_total * BATCH - e
    rt = -(-(n + 1) // (NS * ZCH)) * ZCH
    r_rows = NS * rt
    if pad_e:
        senders = jnp.concatenate(
            [senders, jnp.zeros((pad_e,), jnp.int32)])
        receivers = jnp.concatenate(
            [receivers,
             n + (jnp.arange(pad_e, dtype=jnp.int32) % (r_rows - n))])
    send2d = senders.reshape(nb_total, BATCH)
    recv2d = receivers.reshape(nb_total, BATCH)

    sum_p, deg_p = _sc_aggregate(x.astype(jnp.bfloat16), send2d, recv2d,
                                 r_rows)

    scale = jnp.where(zero_edges, 0.0, 1.0).astype(jnp.float32).reshape(1)
    return _finalize(x, sum_p, deg_p, w_self, b_self, w_neigh, scale)


# speedup vs baseline: 1.0257x; 1.0257x over previous
import jax
import jax.numpy as jnp
from jax import lax
from jax.experimental import pallas as pl
from jax.experimental.pallas import tpu as pltpu
from jax.experimental.pallas import tpu_sc as plsc

NC = 2
NS = 16
NW = NC * NS
BATCH = 128
CH_B = 8
DEG_L = 16
ZCH = 32
NRING = 4
SC0_SHARE = 0.6


def _nbw0(per_pair):
    return int(round(per_pair * SC0_SHARE / (2 * CH_B))) * 2 * CH_B


def _sc_agg_body(x_hbm, send_hbm, recv_hbm, zrow_hbm, zdeg_hbm, ones_hbm,
                 sum_out, deg_out,
                 send_v0, send_v1, recv_v0, recv_v1,
                 rows0, rows1, rows2, rows3, ones_v, zrow_v, zdeg_v,
                 sum_sh, deg_sh,
                 gsem0, gsem1, gsem2, gsem3,
                 ssem0, ssem1, ssem2, ssem3, dsem, isem):
    per_pair = send_hbm.shape[0] // NS
    nbw0 = _nbw0(per_pair)
    nbw1 = per_pair - nbw0
    rt = sum_sh.shape[0] // NS
    c = lax.axis_index("c")
    s = lax.axis_index("s")
    w0 = s * nbw0 + c * (NS * nbw0 + s * (nbw1 - nbw0))
    nch = (nbw0 + c * (nbw1 - nbw0)) // CH_B

    pltpu.sync_copy(zrow_hbm, zrow_v)
    pltpu.sync_copy(zdeg_hbm, zdeg_v)
    pltpu.sync_copy(ones_hbm, ones_v)

    f0 = pltpu.async_copy(send_hbm.at[pl.ds(w0, CH_B), :], send_v0, isem)
    f1 = pltpu.async_copy(recv_hbm.at[pl.ds(w0, CH_B), :], recv_v0, isem)
    zd = []
    for k in range(rt // ZCH):
        zd.append(pltpu.async_copy(
            zrow_v, sum_sh.at[pl.ds(s * rt + k * ZCH, ZCH), :], ssem0))
        zd.append(pltpu.async_copy(
            zdeg_v, deg_sh.at[pl.ds(s * rt + k * ZCH, ZCH), :], ssem1))

    rows = (rows0, rows1, rows2, rows3)
    gsem = (gsem0, gsem1, gsem2, gsem3)
    ssem = (ssem0, ssem1, ssem2, ssem3)

    f0.wait()
    f1.wait()
    gd_pre = []
    for j in range(NRING - 1):
        gd_pre.append(
            pltpu.async_copy(x_hbm.at[send_v0.at[j]], rows[j], gsem[j]))
    for zz in zd:
        zz.wait()
    plsc.subcore_barrier()

    def process_chunk(sv, rv, pre=None):
        gd = [None] * NRING
        sd = [None] * NRING
        dd = []
        for j in range(NRING - 1):
            gd[j] = pre[j] if pre is not None else pltpu.async_copy(
                x_hbm.at[sv.at[j]], rows[j], gsem[j])
        for j in range(CH_B):
            p = j % NRING
            if j + NRING - 1 < CH_B:
                q = (j + NRING - 1) % NRING
                if sd[q] is not None:
                    sd[q].wait()
                gd[q] = pltpu.async_copy(
                    x_hbm.at[sv.at[j + NRING - 1]], rows[q], gsem[q])
            gd[p].wait()
            sd[p] = pltpu.async_copy(
                rows[p], sum_sh.at[rv.at[j]], ssem[p], add=True)
            dd.append(pltpu.async_copy(
                ones_v, deg_sh.at[rv.at[j]], dsem, add=True))
        for p in range(NRING):
            if sd[p] is not None:
                sd[p].wait()
        for d_ in dd:
            d_.wait()

    last = w0 + (nch - 1) * CH_B

    fb0 = pltpu.async_copy(
        send_hbm.at[pl.ds(w0 + CH_B, CH_B), :], send_v1, isem)
    fb1 = pltpu.async_copy(
        recv_hbm.at[pl.ds(w0 + CH_B, CH_B), :], recv_v1, isem)
    process_chunk(send_v0, recv_v0, pre=gd_pre)
    fb0.wait()
    fb1.wait()
    nbase = jnp.minimum(w0 + 2 * CH_B, last)
    fa0 = pltpu.async_copy(send_hbm.at[pl.ds(nbase, CH_B), :], send_v0, isem)
    fa1 = pltpu.async_copy(recv_hbm.at[pl.ds(nbase, CH_B), :], recv_v0, isem)
    process_chunk(send_v1, recv_v1)
    fa0.wait()
    fa1.wait()

    def iter2(i, carry):
        base = w0 + (2 + 2 * i) * CH_B
        fb0 = pltpu.async_copy(
            send_hbm.at[pl.ds(base + CH_B, CH_B), :], send_v1, isem)
        fb1 = pltpu.async_copy(
            recv_hbm.at[pl.ds(base + CH_B, CH_B), :], recv_v1, isem)
        process_chunk(send_v0, recv_v0)
        fb0.wait()
        fb1.wait()
        nbase = jnp.minimum(base + 2 * CH_B, last)
        fa0 = pltpu.async_copy(
            send_hbm.at[pl.ds(nbase, CH_B), :], send_v0, isem)
        fa1 = pltpu.async_copy(
            recv_hbm.at[pl.ds(nbase, CH_B), :], recv_v0, isem)
        process_chunk(send_v1, recv_v1)
        fa0.wait()
        fa1.wait()
        return carry

    lax.fori_loop(0, (nch - 2) // 2, iter2, 0)
    plsc.subcore_barrier()

    pltpu.sync_copy(sum_sh.at[pl.ds(s * rt, rt), :],
                    sum_out.at[c, pl.ds(s * rt, rt), :])
    pltpu.sync_copy(deg_sh.at[pl.ds(s * rt, rt), :],
                    deg_out.at[c, pl.ds(s * rt, rt), :])


def _sc_aggregate(xb, send2d, recv2d, r_rows):
    d = xb.shape[1]
    zrow = jnp.zeros((ZCH, d), jnp.bfloat16)
    zdeg = jnp.zeros((ZCH, DEG_L), jnp.float32)
    ones = jnp.ones((BATCH, DEG_L), jnp.float32)

    mesh = plsc.VectorSubcoreMesh(core_axis_name="c", subcore_axis_name="s")
    f = pl.kernel(
        _sc_agg_body,
        out_type=(
            jax.ShapeDtypeStruct((NC, r_rows, d), jnp.bfloat16),
            jax.ShapeDtypeStruct((NC, r_rows, DEG_L), jnp.float32),
        ),
        mesh=mesh,
        scratch_types=(
            pltpu.VMEM((CH_B, BATCH), jnp.int32),
            pltpu.VMEM((CH_B, BATCH), jnp.int32),
            pltpu.VMEM((CH_B, BATCH), jnp.int32),
            pltpu.VMEM((CH_B, BATCH), jnp.int32),
            pltpu.VMEM((BATCH, d), jnp.bfloat16),
            pltpu.VMEM((BATCH, d), jnp.bfloat16),
            pltpu.VMEM((BATCH, d), jnp.bfloat16),
            pltpu.VMEM((BATCH, d), jnp.bfloat16),
            pltpu.VMEM((BATCH, DEG_L), jnp.float32),
            pltpu.VMEM((ZCH, d), jnp.bfloat16),
            pltpu.VMEM((ZCH, DEG_L), jnp.float32),
            pltpu.VMEM_SHARED((r_rows, d), jnp.bfloat16),
            pltpu.VMEM_SHARED((r_rows, DEG_L), jnp.float32),
            pltpu.SemaphoreType.DMA,
            pltpu.SemaphoreType.DMA,
            pltpu.SemaphoreType.DMA,
            pltpu.SemaphoreType.DMA,
            pltpu.SemaphoreType.DMA,
            pltpu.SemaphoreType.DMA,
            pltpu.SemaphoreType.DMA,
            pltpu.SemaphoreType.DMA,
            pltpu.SemaphoreType.DMA,
            pltpu.SemaphoreType.DMA,
        ),
        compiler_params=pltpu.CompilerParams(use_tc_tiling_on_sc=False),
    )
    return f(xb, send2d, recv2d, zrow, zdeg, ones)


def _finalize_body(x_ref, sum_ref, deg_ref, wsT_ref, wnT_ref, b_ref,
                   scale_ref, out_ref):
    xb = x_ref[...]
    sb = sum_ref[0].astype(jnp.float32) + sum_ref[1].astype(jnp.float32)
    db = deg_ref[0, :, 0:1] + deg_ref[1, :, 0:1]
    recip = jnp.where(db > 0, scale_ref[0] / db, 0.0)
    agg = sb * recip
    acc = lax.dot_general(xb, wsT_ref[...], (((1,), (0,)), ((), ())),
                          precision=lax.Precision.HIGHEST,
                          preferred_element_type=jnp.float32)
    acc = acc + lax.dot_general(agg, wnT_ref[...], (((1,), (0,)), ((), ())),
                                precision=lax.Precision.HIGHEST,
                                preferred_element_type=jnp.float32)
    out_ref[...] = acc + b_ref[...]


def _finalize(x, sum_p, deg_p, w_self, b_self, w_neigh, scale):
    n, d = x.shape
    bn = 1000
    grid = (n // bn,)
    return pl.pallas_call(
        _finalize_body,
        grid=grid,
        in_specs=[
            pl.BlockSpec((bn, d), lambda i: (i, 0)),
            pl.BlockSpec((NC, bn, d), lambda i: (0, i, 0)),
            pl.BlockSpec((NC, bn, DEG_L), lambda i: (0, i, 0)),
            pl.BlockSpec((d, d), lambda i: (0, 0)),
            pl.BlockSpec((d, d), lambda i: (0, 0)),
            pl.BlockSpec((1, d), lambda i: (0, 0)),
            pl.BlockSpec(memory_space=pltpu.SMEM),
        ],
        out_specs=pl.BlockSpec((bn, d), lambda i: (i, 0)),
        out_shape=jax.ShapeDtypeStruct((n, d), jnp.float32),
    )(x, sum_p, deg_p, w_self.T, w_neigh.T, b_self.reshape(1, d), scale)


def kernel(x, senders, receivers, w_self, b_self, w_neigh, zero_edges):
    n, d = x.shape
    e = senders.shape[0]
    senders = senders.astype(jnp.int32)
    receivers = receivers.astype(jnp.int32)

    per_pair = -(-(-(-e // BATCH)) // (NS * 2 * CH_B)) * 2 * CH_B
    nb_total = NS * per_pair
    pad_e = nb_total * BATCH - e
    rt = -(-(n + 1) // (NS * ZCH)) * ZCH
    r_rows = NS * rt
    if pad_e:
        senders = jnp.concatenate(
            [senders, jnp.zeros((pad_e,), jnp.int32)])
        receivers = jnp.concatenate(
            [receivers,
             n + (jnp.arange(pad_e, dtype=jnp.int32) % (r_rows - n))])
    send2d = senders.reshape(nb_total, BATCH)
    recv2d = receivers.reshape(nb_total, BATCH)

    sum_p, deg_p = _sc_aggregate(x.astype(jnp.bfloat16), send2d, recv2d,
                                 r_rows)

    scale = jnp.where(zero_edges, 0.0, 1.0).astype(jnp.float32).reshape(1)
    return _finalize(x, sum_p, deg_p, w_self, b_self, w_neigh, scale)


# speedup vs baseline: 1.0738x; 1.0469x over previous
import jax
import jax.numpy as jnp
from jax import lax
from jax.experimental import pallas as pl
from jax.experimental.pallas import tpu as pltpu
from jax.experimental.pallas import tpu_sc as plsc

NC = 2
NS = 16
NW = NC * NS
BATCH = 128
CH_B = 8
DEG_L = 16
ZCH = 32
NRING = 4
SC0_SHARE = 0.6


def _nbw0(per_pair):
    return int(round(per_pair * SC0_SHARE / (2 * CH_B))) * 2 * CH_B


def _sc_agg_body(x_hbm, send_hbm, recv_hbm, zrow_hbm, zdeg_hbm, ones_hbm,
                 sum_out, deg_out,
                 send_v0, send_v1, recv_v0, recv_v1,
                 rows0, rows1, rows2, rows3, ones_v, zrow_v, zdeg_v,
                 sum_sh, deg_sh,
                 gsem0, gsem1, gsem2, gsem3,
                 ssem0, ssem1, ssem2, ssem3, dsem, isem):
    per_pair = send_hbm.shape[0] // NS
    nbw0 = _nbw0(per_pair)
    nbw1 = per_pair - nbw0
    rt = sum_sh.shape[0] // NS
    c = lax.axis_index("c")
    s = lax.axis_index("s")
    w0 = s * nbw0 + c * (NS * nbw0 + s * (nbw1 - nbw0))
    nch = (nbw0 + c * (nbw1 - nbw0)) // CH_B

    c0 = pltpu.async_copy(zrow_hbm, zrow_v, gsem0)
    c1 = pltpu.async_copy(zdeg_hbm, zdeg_v, gsem1)
    c2 = pltpu.async_copy(ones_hbm, ones_v, gsem2)
    c0.wait()
    c1.wait()
    c2.wait()

    f0 = pltpu.async_copy(send_hbm.at[pl.ds(w0, CH_B), :], send_v0, isem)
    f1 = pltpu.async_copy(recv_hbm.at[pl.ds(w0, CH_B), :], recv_v0, isem)
    zd = []
    for k in range(rt // ZCH):
        zd.append(pltpu.async_copy(
            zrow_v, sum_sh.at[pl.ds(s * rt + k * ZCH, ZCH), :], ssem0))
        zd.append(pltpu.async_copy(
            zdeg_v, deg_sh.at[pl.ds(s * rt + k * ZCH, ZCH), :], ssem1))

    rows = (rows0, rows1, rows2, rows3)
    gsem = (gsem0, gsem1, gsem2, gsem3)
    ssem = (ssem0, ssem1, ssem2, ssem3)

    f0.wait()
    f1.wait()
    gd_pre = []
    for j in range(NRING - 1):
        gd_pre.append(
            pltpu.async_copy(x_hbm.at[send_v0.at[j]], rows[j], gsem[j]))
    for zz in zd:
        zz.wait()
    plsc.subcore_barrier()

    def process_chunk(sv, rv, pre=None):
        gd = [None] * NRING
        sd = [None] * NRING
        dd = []
        for j in range(NRING - 1):
            gd[j] = pre[j] if pre is not None else pltpu.async_copy(
                x_hbm.at[sv.at[j]], rows[j], gsem[j])
        for j in range(CH_B):
            p = j % NRING
            if j + NRING - 1 < CH_B:
                q = (j + NRING - 1) % NRING
                if sd[q] is not None:
                    sd[q].wait()
                gd[q] = pltpu.async_copy(
                    x_hbm.at[sv.at[j + NRING - 1]], rows[q], gsem[q])
            gd[p].wait()
            sd[p] = pltpu.async_copy(
                rows[p], sum_sh.at[rv.at[j]], ssem[p], add=True)
            dd.append(pltpu.async_copy(
                ones_v, deg_sh.at[rv.at[j]], dsem, add=True))
        for p in range(NRING):
            if sd[p] is not None:
                sd[p].wait()
        for d_ in dd:
            d_.wait()

    last = w0 + (nch - 1) * CH_B

    fb0 = pltpu.async_copy(
        send_hbm.at[pl.ds(w0 + CH_B, CH_B), :], send_v1, isem)
    fb1 = pltpu.async_copy(
        recv_hbm.at[pl.ds(w0 + CH_B, CH_B), :], recv_v1, isem)
    process_chunk(send_v0, recv_v0, pre=gd_pre)
    fb0.wait()
    fb1.wait()
    nbase = jnp.minimum(w0 + 2 * CH_B, last)
    fa0 = pltpu.async_copy(send_hbm.at[pl.ds(nbase, CH_B), :], send_v0, isem)
    fa1 = pltpu.async_copy(recv_hbm.at[pl.ds(nbase, CH_B), :], recv_v0, isem)
    process_chunk(send_v1, recv_v1)
    fa0.wait()
    fa1.wait()

    def iter2(i, carry):
        base = w0 + (2 + 2 * i) * CH_B
        fb0 = pltpu.async_copy(
            send_hbm.at[pl.ds(base + CH_B, CH_B), :], send_v1, isem)
        fb1 = pltpu.async_copy(
            recv_hbm.at[pl.ds(base + CH_B, CH_B), :], recv_v1, isem)
        process_chunk(send_v0, recv_v0)
        fb0.wait()
        fb1.wait()
        nbase = jnp.minimum(base + 2 * CH_B, last)
        fa0 = pltpu.async_copy(
            send_hbm.at[pl.ds(nbase, CH_B), :], send_v0, isem)
        fa1 = pltpu.async_copy(
            recv_hbm.at[pl.ds(nbase, CH_B), :], recv_v0, isem)
        process_chunk(send_v1, recv_v1)
        fa0.wait()
        fa1.wait()
        return carry

    lax.fori_loop(0, (nch - 2) // 2, iter2, 0)
    plsc.subcore_barrier()

    p0 = pltpu.async_copy(sum_sh.at[pl.ds(s * rt, rt), :],
                          sum_out.at[c, pl.ds(s * rt, rt), :], gsem0)
    p1 = pltpu.async_copy(deg_sh.at[pl.ds(s * rt, rt), :],
                          deg_out.at[c, pl.ds(s * rt, rt), :], gsem1)
    p0.wait()
    p1.wait()


def _sc_aggregate(xb, send2d, recv2d, r_rows):
    d = xb.shape[1]
    zrow = jnp.zeros((ZCH, d), jnp.bfloat16)
    zdeg = jnp.zeros((ZCH, DEG_L), jnp.float32)
    ones = jnp.ones((BATCH, DEG_L), jnp.float32)

    mesh = plsc.VectorSubcoreMesh(core_axis_name="c", subcore_axis_name="s")
    f = pl.kernel(
        _sc_agg_body,
        out_type=(
            jax.ShapeDtypeStruct((NC, r_rows, d), jnp.bfloat16),
            jax.ShapeDtypeStruct((NC, r_rows, DEG_L), jnp.float32),
        ),
        mesh=mesh,
        scratch_types=(
            pltpu.VMEM((CH_B, BATCH), jnp.int32),
            pltpu.VMEM((CH_B, BATCH), jnp.int32),
            pltpu.VMEM((CH_B, BATCH), jnp.int32),
            pltpu.VMEM((CH_B, BATCH), jnp.int32),
            pltpu.VMEM((BATCH, d), jnp.bfloat16),
            pltpu.VMEM((BATCH, d), jnp.bfloat16),
            pltpu.VMEM((BATCH, d), jnp.bfloat16),
            pltpu.VMEM((BATCH, d), jnp.bfloat16),
            pltpu.VMEM((BATCH, DEG_L), jnp.float32),
            pltpu.VMEM((ZCH, d), jnp.bfloat16),
            pltpu.VMEM((ZCH, DEG_L), jnp.float32),
            pltpu.VMEM_SHARED((r_rows, d), jnp.bfloat16),
            pltpu.VMEM_SHARED((r_rows, DEG_L), jnp.float32),
            pltpu.SemaphoreType.DMA,
            pltpu.SemaphoreType.DMA,
            pltpu.SemaphoreType.DMA,
            pltpu.SemaphoreType.DMA,
            pltpu.SemaphoreType.DMA,
            pltpu.SemaphoreType.DMA,
            pltpu.SemaphoreType.DMA,
            pltpu.SemaphoreType.DMA,
            pltpu.SemaphoreType.DMA,
            pltpu.SemaphoreType.DMA,
        ),
        compiler_params=pltpu.CompilerParams(use_tc_tiling_on_sc=False),
    )
    return f(xb, send2d, recv2d, zrow, zdeg, ones)


def _finalize_body(x_ref, sum_ref, deg_ref, wsT_ref, wnT_ref, b_ref,
                   scale_ref, out_ref):
    xb = x_ref[...]
    sb = sum_ref[0].astype(jnp.float32) + sum_ref[1].astype(jnp.float32)
    db = deg_ref[0, :, 0:1] + deg_ref[1, :, 0:1]
    recip = jnp.where(db > 0, scale_ref[0] / db, 0.0)
    agg = sb * recip
    acc = lax.dot_general(xb, wsT_ref[...], (((1,), (0,)), ((), ())),
                          precision=lax.Precision.HIGHEST,
                          preferred_element_type=jnp.float32)
    acc = acc + lax.dot_general(agg, wnT_ref[...], (((1,), (0,)), ((), ())),
                                precision=lax.Precision.HIGHEST,
                                preferred_element_type=jnp.float32)
    out_ref[...] = acc + b_ref[...]


def _finalize(x, sum_p, deg_p, w_self, b_self, w_neigh, scale):
    n, d = x.shape
    bn = 2000
    grid = (n // bn,)
    return pl.pallas_call(
        _finalize_body,
        grid=grid,
        in_specs=[
            pl.BlockSpec((bn, d), lambda i: (i, 0)),
            pl.BlockSpec((NC, bn, d), lambda i: (0, i, 0)),
            pl.BlockSpec((NC, bn, DEG_L), lambda i: (0, i, 0)),
            pl.BlockSpec((d, d), lambda i: (0, 0)),
            pl.BlockSpec((d, d), lambda i: (0, 0)),
            pl.BlockSpec((1, d), lambda i: (0, 0)),
            pl.BlockSpec(memory_space=pltpu.SMEM),
        ],
        out_specs=pl.BlockSpec((bn, d), lambda i: (i, 0)),
        out_shape=jax.ShapeDtypeStruct((n, d), jnp.float32),
    )(x, sum_p, deg_p, w_self.T, w_neigh.T, b_self.reshape(1, d), scale)


def kernel(x, senders, receivers, w_self, b_self, w_neigh, zero_edges):
    n, d = x.shape
    e = senders.shape[0]
    senders = senders.astype(jnp.int32)
    receivers = receivers.astype(jnp.int32)

    per_pair = -(-(-(-e // BATCH)) // (NS * 2 * CH_B)) * 2 * CH_B
    nb_total = NS * per_pair
    pad_e = nb_total * BATCH - e
    rt = -(-(n + 1) // (NS * ZCH)) * ZCH
    r_rows = NS * rt
    if pad_e:
        senders = jnp.concatenate(
            [senders, jnp.zeros((pad_e,), jnp.int32)])
        receivers = jnp.concatenate(
            [receivers,
             n + (jnp.arange(pad_e, dtype=jnp.int32) % (r_rows - n))])
    send2d = senders.reshape(nb_total, BATCH)
    recv2d = receivers.reshape(nb_total, BATCH)

    sum_p, deg_p = _sc_aggregate(x.astype(jnp.bfloat16), send2d, recv2d,
                                 r_rows)

    scale = jnp.where(zero_edges, 0.0, 1.0).astype(jnp.float32).reshape(1)
    return _finalize(x, sum_p, deg_p, w_self, b_self, w_neigh, scale)
